# Initial kernel scaffold; baseline (speedup 1.0000x reference)
import jax
import jax.numpy as jnp
from jax.experimental import pallas as pl


def kernel(inputs, embed):
    raise NotImplementedError("write your pallas kernel here")


# speedup vs baseline: 4.7047x; 4.7047x over previous
from functools import partial

import jax
import jax.numpy as jnp
from jax.experimental import pallas as pl

N_CODES = 1024
DIM = 64
BLOCK_P = 512

_dot = partial(
    jax.lax.dot_general,
    dimension_numbers=(((1,), (0,)), ((), ())),
    preferred_element_type=jnp.float32,
    precision=jax.lax.Precision.HIGHEST,
)


def _vq_block(x_ref, et_ref, e_ref, idx_ref, q_ref, loss_ref):
    x = x_ref[...]
    et = et_ref[...]
    x2 = x * x
    x3 = x2 * x
    e2t = et * et
    e3t = e2t * et
    c = jnp.sum(e2t * e2t, axis=0, keepdims=True)
    d31 = _dot(x3, et)
    d22 = _dot(x2, e2t)
    d13 = _dot(x, e3t)
    m = (c - 4.0 * (d31 + d13)) + 6.0 * d22
    best = jnp.min(m, axis=1, keepdims=True)
    iota = jax.lax.broadcasted_iota(jnp.int32, m.shape, 1)
    idx = jnp.min(jnp.where(m <= best, iota, N_CODES), axis=1)
    idx_ref[...] = idx[:, None]
    onehot = (iota == idx[:, None]).astype(jnp.float32)
    q = _dot(onehot, e_ref[...])
    diff = q - x
    q_ref[...] = x + diff
    s = jnp.sum(diff * diff, axis=(0, 1), keepdims=True)
    prev = jnp.where(pl.program_id(0) == 0,
                     jnp.zeros((1, 1), jnp.float32), loss_ref[...])
    loss_ref[...] = prev + s


def kernel(inputs, embed):
    inputs = inputs.astype(jnp.float32)
    B, C, H, W = inputs.shape
    x = jnp.transpose(inputs, (0, 2, 3, 1)).reshape(-1, C)
    et = embed.T
    n = x.shape[0]
    grid = (n // BLOCK_P,)
    idx2, q, loss_sum = pl.pallas_call(
        _vq_block,
        grid=grid,
        in_specs=[
            pl.BlockSpec((BLOCK_P, DIM), lambda i: (i, 0)),
            pl.BlockSpec((DIM, N_CODES), lambda i: (0, 0)),
            pl.BlockSpec((N_CODES, DIM), lambda i: (0, 0)),
        ],
        out_specs=[
            pl.BlockSpec((BLOCK_P, 1), lambda i: (i, 0)),
            pl.BlockSpec((BLOCK_P, DIM), lambda i: (i, 0)),
            pl.BlockSpec((1, 1), lambda i: (0, 0)),
        ],
        out_shape=[
            jax.ShapeDtypeStruct((n, 1), jnp.int32),
            jax.ShapeDtypeStruct((n, DIM), jnp.float32),
            jax.ShapeDtypeStruct((1, 1), jnp.float32),
        ],
    )(x, et, embed)
    encoding_indices = idx2.reshape(B, H, W)
    quantized_st = q.reshape(B, H, W, C).transpose(0, 3, 1, 2)
    loss = 0.25 * loss_sum[0, 0] / (B * C * H * W)
    return (quantized_st, encoding_indices, loss)


# speedup vs baseline: 9.7026x; 2.0623x over previous
import functools

import jax
import jax.numpy as jnp
from jax import lax
from jax.experimental import pallas as pl
from jax.experimental.pallas import tpu as pltpu
from jax.experimental.pallas import tpu_sc as plsc

N_CODES = 1024
DIM = 64
BLOCK_P = 1024

SC_NC = 2
SC_NS = 16
SC_NW = SC_NC * SC_NS

_dot = functools.partial(
    jax.lax.dot_general,
    dimension_numbers=(((1,), (0,)), ((), ())),
    preferred_element_type=jnp.float32,
)

KDIM = 4 * DIM


def _dist_block(x_ref, et_ref, i12_ref, b_scr):
    @pl.when(pl.program_id(0) == 0)
    def _():
        et = et_ref[...]
        e2t = et * et
        e3t = e2t * et
        c = jnp.sum(e2t * e2t, axis=0, keepdims=True)
        b = jnp.concatenate(
            [-4.0 * et, -4.0 * e3t, 6.0 * e2t,
             jnp.concatenate([c, jnp.zeros((DIM - 1, N_CODES), jnp.float32)],
                             axis=0)],
            axis=0)
        b_hi = b.astype(jnp.bfloat16)
        b_lo = (b - b_hi.astype(jnp.float32)).astype(jnp.bfloat16)
        b_scr[0:KDIM, :] = b_hi
        b_scr[KDIM:, :] = b_lo

    x = x_ref[...]
    x2 = x * x
    x3 = x2 * x
    a = jnp.concatenate([x3, x, x2, jnp.ones_like(x)], axis=1)
    a_hi = a.astype(jnp.bfloat16)
    a_lo = (a - a_hi.astype(jnp.float32)).astype(jnp.bfloat16)
    b_hi = b_scr[0:KDIM, :]
    b_lo = b_scr[KDIM:, :]
    m = _dot(a_hi, b_hi) + (_dot(a_hi, b_lo) + _dot(a_lo, b_hi))
    iota = lax.broadcasted_iota(jnp.int32, m.shape, 1)
    idx1 = jnp.argmin(m, axis=1).astype(jnp.int32)
    m2 = jnp.where(iota == idx1[:, None], jnp.float32(jnp.inf), m)
    idx2 = jnp.argmin(m2, axis=1).astype(jnp.int32)
    i12_ref[0:BLOCK_P] = idx1[:, None]
    i12_ref[BLOCK_P:] = idx2[:, None]


def _refine_block(x_ref, e1_ref, e2_ref, i1_ref, i2_ref,
                  idx_ref, q_ref, loss_ref):
    x = x_ref[...]
    e1 = e1_ref[:, :DIM]
    e2 = e2_ref[:, :DIM]
    d1 = x - e1
    p1 = d1 * d1
    s1 = jnp.sum(p1 * p1, axis=1, keepdims=True)
    d2 = x - e2
    p2 = d2 * d2
    s2 = jnp.sum(p2 * p2, axis=1, keepdims=True)
    i1 = i1_ref[...]
    i2 = i2_ref[...]
    take2 = (s2 < s1) | ((s2 == s1) & (i2 < i1))
    idx_ref[...] = jnp.where(take2, i2, i1)
    q = jnp.where(take2, e2, e1)
    diff = q - x
    q_ref[...] = x + diff
    s = jnp.sum(diff * diff, axis=(0, 1), keepdims=True)
    prev = jnp.where(pl.program_id(0) == 0,
                     jnp.zeros((1, 1), jnp.float32), loss_ref[...])
    loss_ref[...] = prev + s


def _make_sc_gather(n_rows):
    b_per_w = n_rows // SC_NW
    mesh = plsc.VectorSubcoreMesh(core_axis_name="c", subcore_axis_name="s")

    @functools.partial(
        pl.kernel,
        out_type=jax.ShapeDtypeStruct((n_rows, 2 * DIM), jnp.float32),
        mesh=mesh,
        scratch_types=[
            pltpu.VMEM((b_per_w // 2,), jnp.int32),
            pltpu.VMEM((b_per_w // 2,), jnp.int32),
            pltpu.VMEM((b_per_w // 2, 2 * DIM), jnp.float32),
            pltpu.VMEM((b_per_w // 2, 2 * DIM), jnp.float32),
            pltpu.SemaphoreType.DMA,
            pltpu.SemaphoreType.DMA,
        ],
    )
    def _gather(table_hbm, idx_hbm, out_hbm,
                idx0, idx1, rows0, rows1, sem0, sem1):
        half = b_per_w // 2
        wid = lax.axis_index("s") * SC_NC + lax.axis_index("c")
        base = wid * b_per_w
        pltpu.sync_copy(idx_hbm.at[pl.ds(base, half)], idx0)
        pltpu.sync_copy(idx_hbm.at[pl.ds(base + half, half)], idx1)
        c0 = pltpu.async_copy(table_hbm.at[idx0], rows0, sem0)
        c1 = pltpu.async_copy(table_hbm.at[idx1], rows1, sem1)
        c0.wait()
        pltpu.sync_copy(rows0, out_hbm.at[pl.ds(base, half)])
        c1.wait()
        pltpu.sync_copy(rows1, out_hbm.at[pl.ds(base + half, half)])

    return _gather


def _dist_chunk(x, et, off_blk, nblk_c):
    n_c = nblk_c * BLOCK_P
    return pl.pallas_call(
        _dist_block,
        grid=(nblk_c,),
        in_specs=[
            pl.BlockSpec((BLOCK_P, DIM), lambda i, o=off_blk: (i + o, 0)),
            pl.BlockSpec((DIM, N_CODES), lambda i: (0, 0)),
        ],
        out_specs=pl.BlockSpec((2 * BLOCK_P, 1), lambda i: (i, 0)),
        out_shape=jax.ShapeDtypeStruct((2 * n_c, 1), jnp.int32),
        scratch_shapes=[pltpu.VMEM((2 * KDIM, N_CODES), jnp.bfloat16)],
    )(x, et)


def _refine_chunk(x, rows, i12, off_blk, nblk_c):
    n_c = nblk_c * BLOCK_P
    return pl.pallas_call(
        _refine_block,
        grid=(nblk_c,),
        in_specs=[
            pl.BlockSpec((BLOCK_P, DIM), lambda i, o=off_blk: (i + o, 0)),
            pl.BlockSpec((BLOCK_P, 2 * DIM), lambda i: (2 * i, 0)),
            pl.BlockSpec((BLOCK_P, 2 * DIM), lambda i: (2 * i + 1, 0)),
            pl.BlockSpec((BLOCK_P, 1), lambda i: (2 * i, 0)),
            pl.BlockSpec((BLOCK_P, 1), lambda i: (2 * i + 1, 0)),
        ],
        out_specs=[
            pl.BlockSpec((BLOCK_P, 1), lambda i: (i, 0)),
            pl.BlockSpec((BLOCK_P, DIM), lambda i: (i, 0)),
            pl.BlockSpec((1, 1), lambda i: (0, 0)),
        ],
        out_shape=[
            jax.ShapeDtypeStruct((n_c, 1), jnp.int32),
            jax.ShapeDtypeStruct((n_c, DIM), jnp.float32),
            jax.ShapeDtypeStruct((1, 1), jnp.float32),
        ],
    )(x, rows, rows, i12, i12)


def _chunk_blocks(nblk):
    sizes = [5, 4]
    assert sum(sizes) == nblk
    return sizes


def kernel(inputs, embed):
    inputs = inputs.astype(jnp.float32)
    B, C, H, W = inputs.shape
    x = jnp.transpose(inputs, (0, 2, 3, 1)).reshape(-1, C)
    et = embed.T
    table = jnp.concatenate([embed, jnp.zeros_like(embed)], axis=1)
    n = x.shape[0]
    nblk = n // BLOCK_P

    chunks = []
    off = 0
    for nblk_c in _chunk_blocks(nblk):
        i12_c = _dist_chunk(x, et, off, nblk_c)
        rows_c = _make_sc_gather(2 * nblk_c * BLOCK_P)(table,
                                                       i12_c.reshape(-1))
        chunks.append((i12_c, rows_c, off, nblk_c))
        off += nblk_c

    parts = [_refine_chunk(x, rows_c, i12_c, off_c, nblk_c)
             for i12_c, rows_c, off_c, nblk_c in chunks]
    idx2d = jnp.concatenate([p[0] for p in parts], axis=0)
    q = jnp.concatenate([p[1] for p in parts], axis=0)
    loss_sum = sum(p[2][0, 0] for p in parts)

    encoding_indices = idx2d.reshape(B, H, W)
    quantized_st = q.reshape(B, H, W, C).transpose(0, 3, 1, 2)
    loss = 0.25 * loss_sum / (B * C * H * W)
    return (quantized_st, encoding_indices, loss)
